# baseline (speedup 1.0000x reference)
import functools

import jax
import jax.numpy as jnp
from jax import lax
from jax.experimental import pallas as pl
from jax.experimental.pallas import tpu as pltpu
from jax.experimental.pallas import tpu_sc as plsc

N = 10000
F = 128
FH = F // 2
HID = 512
CPAD = 16
N_PAD = 10240
E = 320000
NCORES = 2
NSUB = 16
NTILES = NCORES * NSUB
BATCH = 1024
NBATCH = 10
EPT = NBATCH * BATCH
E_PAD = NTILES * EPT
BATCH2 = 512
NB2 = 40
EPT2 = NB2 * BATCH2
E_PAD2 = NSUB * EPT2
NBUF = 2
NBUF1 = 1
GB = 4
NGRP2 = NB2 // GB
RPT = N_PAD // NSUB
ZROWS = N_PAD - N


def _mesh():
  return plsc.VectorSubcoreMesh(
      core_axis_name="c", subcore_axis_name="s",
      num_cores=NCORES, num_subcores=NSUB)


def _sc_degree(dst_idx):
  @functools.partial(
      pl.kernel,
      out_type=jax.ShapeDtypeStruct((NCORES, N_PAD), jnp.float32),
      mesh=_mesh(),
      compiler_params=pltpu.CompilerParams(use_tc_tiling_on_sc=False),
      scratch_types=[
          pltpu.VMEM((NBATCH, BATCH), jnp.int32),
          pltpu.VMEM((BATCH,), jnp.float32),
          pltpu.VMEM((RPT,), jnp.float32),
          pltpu.VMEM_SHARED((N_PAD,), jnp.float32),
      ],
  )
  def deg_kernel(dst_hbm, out_hbm, dstv, ones_v, zb, acc):
    c = lax.axis_index("c")
    s = lax.axis_index("s")
    w = c * NSUB + s
    pltpu.sync_copy(dst_hbm.at[w], dstv)
    for i in range(BATCH // 16):
      ones_v[pl.ds(i * 16, 16)] = jnp.ones((16,), jnp.float32)
    for i in range(RPT // 16):
      zb[pl.ds(i * 16, 16)] = jnp.zeros((16,), jnp.float32)
    pltpu.sync_copy(zb, acc.at[pl.ds(s * RPT, RPT)])
    plsc.subcore_barrier()

    def step(j, carry):
      pltpu.sync_copy(ones_v, acc.at[dstv.at[j]], add=True)
      return carry

    lax.fori_loop(0, NBATCH, step, None)
    plsc.subcore_barrier()
    pltpu.sync_copy(acc.at[pl.ds(s * RPT, RPT)],
                    out_hbm.at[c, pl.ds(s * RPT, RPT)])

  return deg_kernel(dst_idx)


def _sc_seg_sum_w128(xs2, src_idx, dst_idx):
  @functools.partial(
      pl.kernel,
      out_type=jax.ShapeDtypeStruct((NCORES, N_PAD, FH), jnp.float32),
      mesh=_mesh(),
      compiler_params=pltpu.CompilerParams(use_tc_tiling_on_sc=False),
      scratch_types=[
          pltpu.VMEM((2 * GB, BATCH2), jnp.int32),
          pltpu.VMEM((2 * GB, BATCH2), jnp.int32),
          pltpu.VMEM((NBUF1, BATCH2, FH), jnp.float32),
          pltpu.VMEM_SHARED((N_PAD, FH), jnp.float32),
          pltpu.VMEM_SHARED((N_PAD, FH), jnp.float32),
          pltpu.SemaphoreType.DMA,
      ],
  )
  def seg_kernel(xs_hbm, src_hbm, dst_hbm, out_hbm, srcg, dstg, rows,
                 xs_sp, acc, *sems):
    c = lax.axis_index("c")
    s = lax.axis_index("s")
    pltpu.sync_copy(dst_hbm.at[s, pl.ds(0, 2 * GB)], dstg)
    pltpu.sync_copy(src_hbm.at[s, pl.ds(0, 2 * GB)], srcg)
    base = s * RPT
    pltpu.sync_copy(xs_hbm.at[c, pl.ds(base, RPT)], xs_sp.at[pl.ds(base, RPT)])
    pltpu.sync_copy(xs_hbm.at[c, pl.ds(N, ZROWS)], acc.at[pl.ds(base, ZROWS)])
    pltpu.sync_copy(xs_hbm.at[c, pl.ds(N, ZROWS)],
                    acc.at[pl.ds(base + ZROWS, ZROWS)])
    pltpu.sync_copy(xs_hbm.at[c, pl.ds(N, RPT - 2 * ZROWS)],
                    acc.at[pl.ds(base + 2 * ZROWS, RPT - 2 * ZROWS)])
    plsc.subcore_barrier()

    for b in range(NBUF1):
      pltpu.async_copy(xs_sp.at[srcg.at[b]], rows.at[b], sems[b])

    def group(g, carry):
      for b in range(GB):
        j = g * GB + b
        buf = b % NBUF1
        pltpu.make_async_copy(xs_hbm.at[c, pl.ds(0, BATCH2)], rows.at[buf],
                              sems[buf]).wait()
        pltpu.sync_copy(rows.at[buf], acc.at[dstg.at[lax.rem(j, 2 * GB)]],
                        add=True)
        jn = j + NBUF1
        jn = jnp.where(jn >= NB2, jn - NB2, jn)
        pltpu.async_copy(xs_sp.at[srcg.at[lax.rem(jn, 2 * GB)]],
                         rows.at[buf], sems[buf])
      gn = lax.rem(g + 2, NGRP2)
      pltpu.sync_copy(src_hbm.at[s, pl.ds(gn * GB, GB)],
                      srcg.at[pl.ds(lax.rem(g, 2) * GB, GB)])
      pltpu.sync_copy(dst_hbm.at[s, pl.ds(gn * GB, GB)],
                      dstg.at[pl.ds(lax.rem(g, 2) * GB, GB)])
      return carry

    lax.fori_loop(0, NGRP2, group, None)
    for b in range(NBUF1):
      pltpu.make_async_copy(xs_hbm.at[c, pl.ds(0, BATCH2)], rows.at[b],
                            sems[b]).wait()
    plsc.subcore_barrier()
    pltpu.sync_copy(acc.at[pl.ds(base, RPT)],
                    out_hbm.at[c, pl.ds(base, RPT)])

  return seg_kernel(xs2, src_idx, dst_idx)


def _sc_seg_sum_w16(ps, src_idx, dst_idx):
  @functools.partial(
      pl.kernel,
      out_type=jax.ShapeDtypeStruct((NCORES, N_PAD, CPAD), jnp.float32),
      mesh=_mesh(),
      compiler_params=pltpu.CompilerParams(use_tc_tiling_on_sc=False),
      scratch_types=[
          pltpu.VMEM((NBATCH, BATCH), jnp.int32),
          pltpu.VMEM((NBATCH, BATCH), jnp.int32),
          pltpu.VMEM((NBUF, BATCH, CPAD), jnp.float32),
          pltpu.VMEM_SHARED((N_PAD, CPAD), jnp.float32),
          pltpu.VMEM_SHARED((N_PAD, CPAD), jnp.float32),
          pltpu.SemaphoreType.DMA,
          pltpu.SemaphoreType.DMA,
      ],
  )
  def seg_kernel(ps_hbm, src_hbm, dst_hbm, out_hbm, srcv, dstv, rows,
                 ps_sp, acc, *sems):
    c = lax.axis_index("c")
    s = lax.axis_index("s")
    w = c * NSUB + s
    pltpu.sync_copy(src_hbm.at[w], srcv)
    pltpu.sync_copy(dst_hbm.at[w], dstv)
    base = s * RPT
    pltpu.sync_copy(ps_hbm.at[pl.ds(base, RPT)], ps_sp.at[pl.ds(base, RPT)])
    pltpu.sync_copy(ps_hbm.at[pl.ds(N, ZROWS)], acc.at[pl.ds(base, ZROWS)])
    pltpu.sync_copy(ps_hbm.at[pl.ds(N, ZROWS)],
                    acc.at[pl.ds(base + ZROWS, ZROWS)])
    pltpu.sync_copy(ps_hbm.at[pl.ds(N, RPT - 2 * ZROWS)],
                    acc.at[pl.ds(base + 2 * ZROWS, RPT - 2 * ZROWS)])
    plsc.subcore_barrier()

    for b in range(NBUF):
      pltpu.async_copy(ps_sp.at[srcv.at[b]], rows.at[b], sems[b])

    def group(g, carry):
      for b in range(NBUF):
        j = g * NBUF + b
        pltpu.make_async_copy(ps_hbm.at[pl.ds(0, BATCH)], rows.at[b],
                              sems[b]).wait()
        pltpu.sync_copy(rows.at[b], acc.at[dstv.at[j]], add=True)
        jn = j + NBUF
        jn = jnp.where(jn >= NBATCH, jn - NBATCH, jn)
        pltpu.async_copy(ps_sp.at[srcv.at[jn]], rows.at[b], sems[b])
      return carry

    lax.fori_loop(0, NBATCH // NBUF, group, None)
    for b in range(NBUF):
      pltpu.make_async_copy(ps_hbm.at[pl.ds(0, BATCH)], rows.at[b],
                            sems[b]).wait()
    plsc.subcore_barrier()
    pltpu.sync_copy(acc.at[pl.ds(base, RPT)],
                    out_hbm.at[c, pl.ds(base, RPT)])

  return seg_kernel(ps, src_idx, dst_idx)


_ROWBLK = 2560
_GRID = N_PAD // _ROWBLK


def _tc_prescale(deg_col, x):
  def body(deg_ref, x_ref, dinv_ref, xs_ref):
    dinv = lax.rsqrt(deg_ref[...])
    dinv_ref[...] = dinv
    xs = x_ref[...] * dinv
    xs_ref[0] = xs[:, :FH]
    xs_ref[1] = xs[:, FH:]

  return pl.pallas_call(
      body,
      grid=(_GRID,),
      in_specs=[
          pl.BlockSpec((_ROWBLK, 1), lambda i: (i, 0)),
          pl.BlockSpec((_ROWBLK, F), lambda i: (i, 0)),
      ],
      out_specs=[
          pl.BlockSpec((_ROWBLK, 1), lambda i: (i, 0)),
          pl.BlockSpec((NCORES, _ROWBLK, FH), lambda i: (0, i, 0)),
      ],
      out_shape=[
          jax.ShapeDtypeStruct((N_PAD, 1), jnp.float32),
          jax.ShapeDtypeStruct((NCORES, N_PAD, FH), jnp.float32),
      ],
  )(deg_col, x)


def _tc_layers(y, xs2, dinv_col, w1, b1, w2p):
  def body(y0_ref, y1_ref, xs0_ref, xs1_ref, dinv_ref, w1_ref, b1_ref,
           w2_ref, ps_ref):
    dinv = dinv_ref[...]
    agg = jnp.concatenate(
        [y0_ref[0] + xs0_ref[0], y1_ref[0] + xs1_ref[0]], axis=1) * dinv
    h = jnp.dot(agg, w1_ref[...], preferred_element_type=jnp.float32)
    h = jnp.maximum(h + b1_ref[...], 0.0)
    p = jnp.dot(h, w2_ref[...], preferred_element_type=jnp.float32)
    rid = lax.broadcasted_iota(jnp.int32, (_ROWBLK, 1), 0)
    rid = rid + pl.program_id(0) * _ROWBLK
    ps_ref[...] = jnp.where(rid < N, p * dinv, 0.0)

  return pl.pallas_call(
      body,
      grid=(_GRID,),
      in_specs=[
          pl.BlockSpec((1, _ROWBLK, FH), lambda i: (0, i, 0)),
          pl.BlockSpec((1, _ROWBLK, FH), lambda i: (1, i, 0)),
          pl.BlockSpec((1, _ROWBLK, FH), lambda i: (0, i, 0)),
          pl.BlockSpec((1, _ROWBLK, FH), lambda i: (1, i, 0)),
          pl.BlockSpec((_ROWBLK, 1), lambda i: (i, 0)),
          pl.BlockSpec((F, HID), lambda i: (0, 0)),
          pl.BlockSpec((1, HID), lambda i: (0, 0)),
          pl.BlockSpec((HID, CPAD), lambda i: (0, 0)),
      ],
      out_specs=pl.BlockSpec((_ROWBLK, CPAD), lambda i: (i, 0)),
      out_shape=jax.ShapeDtypeStruct((N_PAD, CPAD), jnp.float32),
  )(y, y, xs2, xs2, dinv_col, w1, b1, w2p)


def _tc_softmax(y2, ps, dinv_col, b2p):
  def body(y0_ref, y1_ref, ps_ref, dinv_ref, b2_ref, out_ref):
    z = (y0_ref[0] + y1_ref[0] + ps_ref[...]) * dinv_ref[...] + b2_ref[...]
    z0 = z[:, 0:1]
    z1 = z[:, 1:2]
    m = jnp.maximum(z0, z1)
    e0 = jnp.exp(z0 - m)
    e1 = jnp.exp(z1 - m)
    inv = 1.0 / (e0 + e1)
    out_ref[...] = jnp.concatenate([e0 * inv, e1 * inv], axis=1)

  return pl.pallas_call(
      body,
      grid=(_GRID,),
      in_specs=[
          pl.BlockSpec((1, _ROWBLK, CPAD), lambda i: (0, i, 0)),
          pl.BlockSpec((1, _ROWBLK, CPAD), lambda i: (1, i, 0)),
          pl.BlockSpec((_ROWBLK, CPAD), lambda i: (i, 0)),
          pl.BlockSpec((_ROWBLK, 1), lambda i: (i, 0)),
          pl.BlockSpec((1, CPAD), lambda i: (0, 0)),
      ],
      out_specs=pl.BlockSpec((_ROWBLK, 2), lambda i: (i, 0)),
      out_shape=jax.ShapeDtypeStruct((N_PAD, 2), jnp.float32),
  )(y2, y2, ps, dinv_col, b2p)


def kernel(X, edge_index, W1, b1, W2, b2):
  src = edge_index[0].astype(jnp.int32)
  dst = edge_index[1].astype(jnp.int32)
  pad = jnp.full((E_PAD - E,), N, jnp.int32)
  srcp = jnp.concatenate([src, pad]).reshape(NTILES, NBATCH, BATCH)
  dstp = jnp.concatenate([dst, pad]).reshape(NTILES, NBATCH, BATCH)
  pad2 = jnp.full((E_PAD2 - E,), N, jnp.int32)
  srcp2 = jnp.concatenate([src, pad2]).reshape(NSUB, NB2, BATCH2)
  dstp2 = jnp.concatenate([dst, pad2]).reshape(NSUB, NB2, BATCH2)
  xp = jnp.concatenate([X, jnp.zeros((N_PAD - N, F), X.dtype)], axis=0)

  degpart = _sc_degree(dstp)
  deg_col = (degpart[0] + degpart[1] + 1.0)[:, None]
  dinv_col, xs2 = _tc_prescale(deg_col, xp)

  y = _sc_seg_sum_w128(xs2, srcp2, dstp2)

  w2p = jnp.pad(W2, ((0, 0), (0, CPAD - W2.shape[1])))
  b2p = jnp.pad(b2, (0, CPAD - b2.shape[0]))[None, :]
  ps = _tc_layers(y, xs2, dinv_col, W1, b1[None, :], w2p)

  y2 = _sc_seg_sum_w16(ps, srcp, dstp)
  out = _tc_softmax(y2, ps, dinv_col, b2p)
  return out[:N]


# speedup vs baseline: 1.0296x; 1.0296x over previous
import functools

import jax
import jax.numpy as jnp
from jax import lax
from jax.experimental import pallas as pl
from jax.experimental.pallas import tpu as pltpu
from jax.experimental.pallas import tpu_sc as plsc

N = 10000
F = 128
FH = F // 2
HID = 512
CPAD = 16
N_PAD = 10240
E = 320000
NCORES = 2
NSUB = 16
NTILES = NCORES * NSUB
BATCH = 1024
NBATCH = 10
EPT = NBATCH * BATCH
E_PAD = NTILES * EPT
BATCH2 = 256
NB2 = 80
EPT2 = NB2 * BATCH2
E_PAD2 = NSUB * EPT2
NBUF = 2
NBUF1 = 2
GB = 8
NGRP2 = NB2 // GB
RPT = N_PAD // NSUB
ZROWS = N_PAD - N


def _mesh():
  return plsc.VectorSubcoreMesh(
      core_axis_name="c", subcore_axis_name="s",
      num_cores=NCORES, num_subcores=NSUB)


def _sc_degree(dst_idx):
  @functools.partial(
      pl.kernel,
      out_type=jax.ShapeDtypeStruct((NCORES, N_PAD), jnp.float32),
      mesh=_mesh(),
      compiler_params=pltpu.CompilerParams(use_tc_tiling_on_sc=False),
      scratch_types=[
          pltpu.VMEM((NBATCH, BATCH), jnp.int32),
          pltpu.VMEM((BATCH,), jnp.float32),
          pltpu.VMEM((RPT,), jnp.float32),
          pltpu.VMEM_SHARED((N_PAD,), jnp.float32),
      ],
  )
  def deg_kernel(dst_hbm, out_hbm, dstv, ones_v, zb, acc):
    c = lax.axis_index("c")
    s = lax.axis_index("s")
    w = c * NSUB + s
    pltpu.sync_copy(dst_hbm.at[w], dstv)
    for i in range(BATCH // 16):
      ones_v[pl.ds(i * 16, 16)] = jnp.ones((16,), jnp.float32)
    for i in range(RPT // 16):
      zb[pl.ds(i * 16, 16)] = jnp.zeros((16,), jnp.float32)
    pltpu.sync_copy(zb, acc.at[pl.ds(s * RPT, RPT)])
    plsc.subcore_barrier()

    def step(j, carry):
      pltpu.sync_copy(ones_v, acc.at[dstv.at[j]], add=True)
      return carry

    lax.fori_loop(0, NBATCH, step, None)
    plsc.subcore_barrier()
    pltpu.sync_copy(acc.at[pl.ds(s * RPT, RPT)],
                    out_hbm.at[c, pl.ds(s * RPT, RPT)])

  return deg_kernel(dst_idx)


def _sc_seg_sum_w128(xs2, src_idx, dst_idx):
  @functools.partial(
      pl.kernel,
      out_type=jax.ShapeDtypeStruct((NCORES, N_PAD, FH), jnp.float32),
      mesh=_mesh(),
      compiler_params=pltpu.CompilerParams(use_tc_tiling_on_sc=False),
      scratch_types=[
          pltpu.VMEM((2 * GB, BATCH2), jnp.int32),
          pltpu.VMEM((2 * GB, BATCH2), jnp.int32),
          pltpu.VMEM((NBUF1, BATCH2, FH), jnp.float32),
          pltpu.VMEM_SHARED((N_PAD, FH), jnp.float32),
          pltpu.VMEM_SHARED((N_PAD, FH), jnp.float32),
          pltpu.SemaphoreType.DMA,
          pltpu.SemaphoreType.DMA,
      ],
  )
  def seg_kernel(xs_hbm, src_hbm, dst_hbm, out_hbm, srcg, dstg, rows,
                 xs_sp, acc, *sems):
    c = lax.axis_index("c")
    s = lax.axis_index("s")
    pltpu.sync_copy(dst_hbm.at[s, pl.ds(0, 2 * GB)], dstg)
    pltpu.sync_copy(src_hbm.at[s, pl.ds(0, 2 * GB)], srcg)
    base = s * RPT
    pltpu.sync_copy(xs_hbm.at[c, pl.ds(base, RPT)], xs_sp.at[pl.ds(base, RPT)])
    pltpu.sync_copy(xs_hbm.at[c, pl.ds(N, ZROWS)], acc.at[pl.ds(base, ZROWS)])
    pltpu.sync_copy(xs_hbm.at[c, pl.ds(N, ZROWS)],
                    acc.at[pl.ds(base + ZROWS, ZROWS)])
    pltpu.sync_copy(xs_hbm.at[c, pl.ds(N, RPT - 2 * ZROWS)],
                    acc.at[pl.ds(base + 2 * ZROWS, RPT - 2 * ZROWS)])
    plsc.subcore_barrier()

    for b in range(NBUF1):
      pltpu.async_copy(xs_sp.at[srcg.at[b]], rows.at[b], sems[b])

    def group(g, carry):
      for b in range(GB):
        j = g * GB + b
        buf = b % NBUF1
        pltpu.make_async_copy(xs_hbm.at[c, pl.ds(0, BATCH2)], rows.at[buf],
                              sems[buf]).wait()
        pltpu.sync_copy(rows.at[buf], acc.at[dstg.at[lax.rem(j, 2 * GB)]],
                        add=True)
        jn = j + NBUF1
        jn = jnp.where(jn >= NB2, jn - NB2, jn)
        pltpu.async_copy(xs_sp.at[srcg.at[lax.rem(jn, 2 * GB)]],
                         rows.at[buf], sems[buf])
      gn = lax.rem(g + 2, NGRP2)
      pltpu.sync_copy(src_hbm.at[s, pl.ds(gn * GB, GB)],
                      srcg.at[pl.ds(lax.rem(g, 2) * GB, GB)])
      pltpu.sync_copy(dst_hbm.at[s, pl.ds(gn * GB, GB)],
                      dstg.at[pl.ds(lax.rem(g, 2) * GB, GB)])
      return carry

    lax.fori_loop(0, NGRP2, group, None)
    for b in range(NBUF1):
      pltpu.make_async_copy(xs_hbm.at[c, pl.ds(0, BATCH2)], rows.at[b],
                            sems[b]).wait()
    plsc.subcore_barrier()
    pltpu.sync_copy(acc.at[pl.ds(base, RPT)],
                    out_hbm.at[c, pl.ds(base, RPT)])

  return seg_kernel(xs2, src_idx, dst_idx)


def _sc_seg_sum_w16(ps, src_idx, dst_idx):
  @functools.partial(
      pl.kernel,
      out_type=jax.ShapeDtypeStruct((NCORES, N_PAD, CPAD), jnp.float32),
      mesh=_mesh(),
      compiler_params=pltpu.CompilerParams(use_tc_tiling_on_sc=False),
      scratch_types=[
          pltpu.VMEM((NBATCH, BATCH), jnp.int32),
          pltpu.VMEM((NBATCH, BATCH), jnp.int32),
          pltpu.VMEM((NBUF, BATCH, CPAD), jnp.float32),
          pltpu.VMEM_SHARED((N_PAD, CPAD), jnp.float32),
          pltpu.VMEM_SHARED((N_PAD, CPAD), jnp.float32),
          pltpu.SemaphoreType.DMA,
          pltpu.SemaphoreType.DMA,
      ],
  )
  def seg_kernel(ps_hbm, src_hbm, dst_hbm, out_hbm, srcv, dstv, rows,
                 ps_sp, acc, *sems):
    c = lax.axis_index("c")
    s = lax.axis_index("s")
    w = c * NSUB + s
    pltpu.sync_copy(src_hbm.at[w], srcv)
    pltpu.sync_copy(dst_hbm.at[w], dstv)
    base = s * RPT
    pltpu.sync_copy(ps_hbm.at[pl.ds(base, RPT)], ps_sp.at[pl.ds(base, RPT)])
    pltpu.sync_copy(ps_hbm.at[pl.ds(N, ZROWS)], acc.at[pl.ds(base, ZROWS)])
    pltpu.sync_copy(ps_hbm.at[pl.ds(N, ZROWS)],
                    acc.at[pl.ds(base + ZROWS, ZROWS)])
    pltpu.sync_copy(ps_hbm.at[pl.ds(N, RPT - 2 * ZROWS)],
                    acc.at[pl.ds(base + 2 * ZROWS, RPT - 2 * ZROWS)])
    plsc.subcore_barrier()

    for b in range(NBUF):
      pltpu.async_copy(ps_sp.at[srcv.at[b]], rows.at[b], sems[b])

    def group(g, carry):
      for b in range(NBUF):
        j = g * NBUF + b
        pltpu.make_async_copy(ps_hbm.at[pl.ds(0, BATCH)], rows.at[b],
                              sems[b]).wait()
        pltpu.sync_copy(rows.at[b], acc.at[dstv.at[j]], add=True)
        jn = j + NBUF
        jn = jnp.where(jn >= NBATCH, jn - NBATCH, jn)
        pltpu.async_copy(ps_sp.at[srcv.at[jn]], rows.at[b], sems[b])
      return carry

    lax.fori_loop(0, NBATCH // NBUF, group, None)
    for b in range(NBUF):
      pltpu.make_async_copy(ps_hbm.at[pl.ds(0, BATCH)], rows.at[b],
                            sems[b]).wait()
    plsc.subcore_barrier()
    pltpu.sync_copy(acc.at[pl.ds(base, RPT)],
                    out_hbm.at[c, pl.ds(base, RPT)])

  return seg_kernel(ps, src_idx, dst_idx)


_ROWBLK = 2560
_GRID = N_PAD // _ROWBLK


def _tc_prescale(deg_col, x):
  def body(deg_ref, x_ref, dinv_ref, xs_ref):
    dinv = lax.rsqrt(deg_ref[...])
    dinv_ref[...] = dinv
    xs = x_ref[...] * dinv
    xs_ref[0] = xs[:, :FH]
    xs_ref[1] = xs[:, FH:]

  return pl.pallas_call(
      body,
      grid=(_GRID,),
      in_specs=[
          pl.BlockSpec((_ROWBLK, 1), lambda i: (i, 0)),
          pl.BlockSpec((_ROWBLK, F), lambda i: (i, 0)),
      ],
      out_specs=[
          pl.BlockSpec((_ROWBLK, 1), lambda i: (i, 0)),
          pl.BlockSpec((NCORES, _ROWBLK, FH), lambda i: (0, i, 0)),
      ],
      out_shape=[
          jax.ShapeDtypeStruct((N_PAD, 1), jnp.float32),
          jax.ShapeDtypeStruct((NCORES, N_PAD, FH), jnp.float32),
      ],
  )(deg_col, x)


def _tc_layers(y, xs2, dinv_col, w1, b1, w2p):
  def body(y0_ref, y1_ref, xs0_ref, xs1_ref, dinv_ref, w1_ref, b1_ref,
           w2_ref, ps_ref):
    dinv = dinv_ref[...]
    agg = jnp.concatenate(
        [y0_ref[0] + xs0_ref[0], y1_ref[0] + xs1_ref[0]], axis=1) * dinv
    h = jnp.dot(agg, w1_ref[...], preferred_element_type=jnp.float32)
    h = jnp.maximum(h + b1_ref[...], 0.0)
    p = jnp.dot(h, w2_ref[...], preferred_element_type=jnp.float32)
    rid = lax.broadcasted_iota(jnp.int32, (_ROWBLK, 1), 0)
    rid = rid + pl.program_id(0) * _ROWBLK
    ps_ref[...] = jnp.where(rid < N, p * dinv, 0.0)

  return pl.pallas_call(
      body,
      grid=(_GRID,),
      in_specs=[
          pl.BlockSpec((1, _ROWBLK, FH), lambda i: (0, i, 0)),
          pl.BlockSpec((1, _ROWBLK, FH), lambda i: (1, i, 0)),
          pl.BlockSpec((1, _ROWBLK, FH), lambda i: (0, i, 0)),
          pl.BlockSpec((1, _ROWBLK, FH), lambda i: (1, i, 0)),
          pl.BlockSpec((_ROWBLK, 1), lambda i: (i, 0)),
          pl.BlockSpec((F, HID), lambda i: (0, 0)),
          pl.BlockSpec((1, HID), lambda i: (0, 0)),
          pl.BlockSpec((HID, CPAD), lambda i: (0, 0)),
      ],
      out_specs=pl.BlockSpec((_ROWBLK, CPAD), lambda i: (i, 0)),
      out_shape=jax.ShapeDtypeStruct((N_PAD, CPAD), jnp.float32),
  )(y, y, xs2, xs2, dinv_col, w1, b1, w2p)


def _tc_softmax(y2, ps, dinv_col, b2p):
  def body(y0_ref, y1_ref, ps_ref, dinv_ref, b2_ref, out_ref):
    z = (y0_ref[0] + y1_ref[0] + ps_ref[...]) * dinv_ref[...] + b2_ref[...]
    z0 = z[:, 0:1]
    z1 = z[:, 1:2]
    m = jnp.maximum(z0, z1)
    e0 = jnp.exp(z0 - m)
    e1 = jnp.exp(z1 - m)
    inv = 1.0 / (e0 + e1)
    out_ref[...] = jnp.concatenate([e0 * inv, e1 * inv], axis=1)

  return pl.pallas_call(
      body,
      grid=(_GRID,),
      in_specs=[
          pl.BlockSpec((1, _ROWBLK, CPAD), lambda i: (0, i, 0)),
          pl.BlockSpec((1, _ROWBLK, CPAD), lambda i: (1, i, 0)),
          pl.BlockSpec((_ROWBLK, CPAD), lambda i: (i, 0)),
          pl.BlockSpec((_ROWBLK, 1), lambda i: (i, 0)),
          pl.BlockSpec((1, CPAD), lambda i: (0, 0)),
      ],
      out_specs=pl.BlockSpec((_ROWBLK, 2), lambda i: (i, 0)),
      out_shape=jax.ShapeDtypeStruct((N_PAD, 2), jnp.float32),
  )(y2, y2, ps, dinv_col, b2p)


def kernel(X, edge_index, W1, b1, W2, b2):
  src = edge_index[0].astype(jnp.int32)
  dst = edge_index[1].astype(jnp.int32)
  pad = jnp.full((E_PAD - E,), N, jnp.int32)
  srcp = jnp.concatenate([src, pad]).reshape(NTILES, NBATCH, BATCH)
  dstp = jnp.concatenate([dst, pad]).reshape(NTILES, NBATCH, BATCH)
  pad2 = jnp.full((E_PAD2 - E,), N, jnp.int32)
  srcp2 = jnp.concatenate([src, pad2]).reshape(NSUB, NB2, BATCH2)
  dstp2 = jnp.concatenate([dst, pad2]).reshape(NSUB, NB2, BATCH2)
  xp = jnp.concatenate([X, jnp.zeros((N_PAD - N, F), X.dtype)], axis=0)

  degpart = _sc_degree(dstp)
  deg_col = (degpart[0] + degpart[1] + 1.0)[:, None]
  dinv_col, xs2 = _tc_prescale(deg_col, xp)

  y = _sc_seg_sum_w128(xs2, srcp2, dstp2)

  w2p = jnp.pad(W2, ((0, 0), (0, CPAD - W2.shape[1])))
  b2p = jnp.pad(b2, (0, CPAD - b2.shape[0]))[None, :]
  ps = _tc_layers(y, xs2, dinv_col, W1, b1[None, :], w2p)

  y2 = _sc_seg_sum_w16(ps, srcp, dstp)
  out = _tc_softmax(y2, ps, dinv_col, b2p)
  return out[:N]
